# baseline (speedup 1.0000x reference)
import functools

import jax
import jax.numpy as jnp
import numpy as np
from jax import lax
from jax.experimental import pallas as pl
from jax.experimental.pallas import tpu as pltpu
from jax.experimental.pallas import tpu_sc as plsc

_TINY = np.float32(np.finfo(np.float32).tiny)
_KS0 = np.uint32(0)
_KS1 = np.uint32(42)
_KS2 = np.uint32(_KS0 ^ _KS1 ^ np.uint32(0x1BD11BDA))
_ROT_A = (13, 15, 26, 6)
_ROT_B = (17, 29, 16, 24)
_LANES = 128


def _rotl(x, r):
    return jax.lax.shift_left(x, np.uint32(r)) | jax.lax.shift_right_logical(
        x, np.uint32(32 - r))


def _threefry_bits(x1):
    x0 = x1
    x1 = _rotl(x1, _ROT_A[0]) ^ x0
    for r in _ROT_A[1:]:
        x0 = x0 + x1
        x1 = _rotl(x1, r) ^ x0
    x0 = x0 + _KS1
    x1 = x1 + np.uint32(_KS2 + np.uint32(1))
    for r in _ROT_B:
        x0 = x0 + x1
        x1 = _rotl(x1, r) ^ x0
    x0 = x0 + _KS2
    x1 = x1 + np.uint32(_KS0 + np.uint32(2))
    for r in _ROT_A:
        x0 = x0 + x1
        x1 = _rotl(x1, r) ^ x0
    x0 = x0 + _KS0
    x1 = x1 + np.uint32(_KS1 + np.uint32(3))
    for r in _ROT_B:
        x0 = x0 + x1
        x1 = _rotl(x1, r) ^ x0
    x0 = x0 + _KS1
    x1 = x1 + np.uint32(_KS2 + np.uint32(4))
    for r in _ROT_A:
        x0 = x0 + x1
        x1 = _rotl(x1, r) ^ x0
    x0 = x0 + _KS2
    x1 = x1 + np.uint32(_KS0 + np.uint32(5))
    return x0 ^ x1


def _uniform_from_bits(bits):
    fb = jax.lax.shift_right_logical(bits, np.uint32(9)) | np.uint32(0x3F800000)
    f = jax.lax.bitcast_convert_type(fb, jnp.float32)
    return jnp.maximum(_TINY, (f - np.float32(1.0)) + _TINY)


_SW = 512


def _tc_kernel(logits_ref, m_out, z_out, i_out, m_ref, z_ref, tc_ref, wc_ref,
               i_ref, *, width, vocab, rows, nblocks, col0):
    j = pl.program_id(0)

    @pl.when(j == 0)
    def _init():
        m_ref[...] = jnp.full((rows, _SW), -np.inf, dtype=jnp.float32)
        z_ref[...] = jnp.zeros((rows, _SW), dtype=jnp.float32)
        tc_ref[...] = jnp.zeros((rows, _SW), dtype=jnp.float32)
        wc_ref[...] = jnp.full((rows, _SW), -1.0, dtype=jnp.float32)
        i_ref[...] = jnp.zeros((rows, _SW), dtype=jnp.int32)

    m_acc = m_ref[...]
    z_acc = z_ref[...]
    tc_acc = tc_ref[...]
    wc_acc = wc_ref[...]
    i_acc = i_ref[...]
    for k in range(width // _SW):
        xk_raw = logits_ref[:, k * _SW:(k + 1) * _SW]
        shape = xk_raw.shape
        col = (col0 + j * width + k * _SW
               + jax.lax.broadcasted_iota(jnp.int32, shape, 1))
        xk = jnp.where(col < vocab, xk_raw, -jnp.inf)
        row = jax.lax.broadcasted_iota(jnp.int32, shape, 0)
        ctr = (row * vocab + col + jnp.int32(_KS1)).astype(jnp.uint32)
        u = _uniform_from_bits(_threefry_bits(ctr))
        w = jnp.log(u)

        m_new = jnp.maximum(m_acc, xk)
        resc = jnp.exp(m_acc - m_new)
        t = jnp.exp(xk - m_new)
        z_acc = z_acc * resc + t
        tc_resc = tc_acc * resc
        better = (t * wc_acc) < (tc_resc * w)
        tc_acc = jnp.where(better, t, tc_resc)
        wc_acc = jnp.where(better, w, wc_acc)
        i_acc = jnp.where(better, col, i_acc)
        m_acc = m_new
    m_ref[...] = m_acc
    z_ref[...] = z_acc
    tc_ref[...] = tc_acc
    wc_ref[...] = wc_acc
    i_ref[...] = i_acc

    @pl.when(j == nblocks - 1)
    def _fin():
        m_out[...] = m_acc
        z_out[...] = z_acc
        i_out[...] = i_acc


def _run_tc(logits, width, col0=0):
    rows, vocab = logits.shape
    nblocks = (vocab - col0 + width - 1) // width
    boff = col0 // width
    kern = functools.partial(_tc_kernel, width=width, vocab=vocab, rows=rows,
                             nblocks=nblocks, col0=col0)
    acc = pl.pallas_call(
        kern,
        grid=(nblocks,),
        in_specs=[pl.BlockSpec((rows, width), lambda j: (0, j + boff))],
        out_specs=[pl.BlockSpec((rows, _SW), lambda j: (0, 0))] * 3,
        out_shape=[jax.ShapeDtypeStruct((rows, _SW), jnp.float32),
                   jax.ShapeDtypeStruct((rows, _SW), jnp.float32),
                   jax.ShapeDtypeStruct((rows, _SW), jnp.int32)],
        scratch_shapes=[pltpu.VMEM((rows, _SW), jnp.float32),
                        pltpu.VMEM((rows, _SW), jnp.float32),
                        pltpu.VMEM((rows, _SW), jnp.float32),
                        pltpu.VMEM((rows, _SW), jnp.float32),
                        pltpu.VMEM((rows, _SW), jnp.int32)],
        compiler_params=pltpu.CompilerParams(
            dimension_semantics=("arbitrary",)),
    )(logits)
    return acc


_CH = 28672
_LN2 = np.float32(0.6931471805599453)
_LOG1P_C = [np.float32((1.0 if k % 2 else -1.0) / k) for k in range(11, 0, -1)]


def _softlog(u):
    i = jax.lax.bitcast_convert_type(u, jnp.int32)
    e = jax.lax.shift_right_arithmetic(i, jnp.int32(23)) - jnp.int32(127)
    mb = (i & jnp.int32(0x007FFFFF)) | jnp.int32(0x3F800000)
    m = jax.lax.bitcast_convert_type(mb, jnp.float32)
    big = mb > jnp.int32(0x3FB504F3)
    m2 = jnp.where(big, m * np.float32(0.5), m)
    e2 = jnp.where(big, e + jnp.int32(1), e)
    y = m2 - np.float32(1.0)
    p = jnp.full_like(y, _LOG1P_C[0])
    for cns in _LOG1P_C[1:]:
        p = p * y + cns
    return e2.astype(jnp.float32) * _LN2 + y * p


def _make_sc_shard(rows, vocab, csc):
    nchunks = csc // _CH
    mesh = plsc.VectorSubcoreMesh(core_axis_name="c", subcore_axis_name="s")

    @functools.partial(
        pl.kernel, mesh=mesh,
        out_type=[jax.ShapeDtypeStruct((rows, 32), jnp.float32),
                  jax.ShapeDtypeStruct((rows, 32), jnp.float32),
                  jax.ShapeDtypeStruct((rows, 32), jnp.int32)],
        scratch_types=[pltpu.VMEM((2, _CH), jnp.float32),
                       pltpu.VMEM((32,), jnp.float32),
                       pltpu.VMEM((32,), jnp.float32),
                       pltpu.VMEM((32,), jnp.int32),
                       pltpu.SemaphoreType.DMA,
                       pltpu.SemaphoreType.DMA],
    )
    def sc_kern(logits_hbm, m_out, z_out, i_out, buf, mf_st, zf_st, if_st,
                sem0, sem1):
        wid = lax.axis_index("s") * 2 + lax.axis_index("c")
        rowbase = wid * jnp.int32(vocab) + jnp.int32(int(_KS1))
        sems = (sem0, sem1)
        copies = [None, None]
        copies[0] = pltpu.async_copy(
            logits_hbm.at[wid, pl.ds(0, _CH)], buf.at[0], sems[0])
        lane = lax.iota(jnp.int32, 16)

        def chunk_body(carry, cidx, b):
            def step(i, st):
                st0, st1 = st
                new = []
                for half, sth in ((0, st0), (1, st1)):
                    m_a, z_a, tc_a, wc_a, ic_a = sth
                    xk = buf[b, pl.ds(i * 32 + half * 16, 16)]
                    col = (jnp.int32(cidx * _CH) + i * 32 + half * 16) + lane
                    ctr = (rowbase + col).astype(jnp.uint32)
                    u = _uniform_from_bits(_threefry_bits(ctr))
                    w = _softlog(u)
                    m_new = jnp.maximum(m_a, xk)
                    resc = jnp.exp(m_a - m_new)
                    t = jnp.exp(xk - m_new)
                    z_a = z_a * resc + t
                    tc_r = tc_a * resc
                    better = (t * wc_a) < (tc_r * w)
                    tc_a = jnp.where(better, t, tc_r)
                    wc_a = jnp.where(better, w, wc_a)
                    ic_a = jnp.where(better, col, ic_a)
                    new.append((m_new, z_a, tc_a, wc_a, ic_a))
                return (new[0], new[1])

            return lax.fori_loop(0, _CH // 32, step, carry)

        half_init = (jnp.full((16,), np.float32(-3.0e38), jnp.float32),
                     jnp.zeros((16,), jnp.float32),
                     jnp.zeros((16,), jnp.float32),
                     jnp.full((16,), np.float32(-1.0), jnp.float32),
                     jnp.zeros((16,), jnp.int32))
        carry = (half_init, half_init)
        for c in range(nchunks):
            b = c % 2
            if c + 1 < nchunks:
                copies[1 - b] = pltpu.async_copy(
                    logits_hbm.at[wid, pl.ds((c + 1) * _CH, _CH)],
                    buf.at[1 - b], sems[1 - b])
            copies[b].wait()
            carry = chunk_body(carry, c, b)
        for half in (0, 1):
            m_a, z_a, tc_a, wc_a, ic_a = carry[half]
            sl = pl.ds(half * 16, 16)
            mf_st[sl] = m_a
            zf_st[sl] = z_a
            if_st[sl] = ic_a
        pltpu.sync_copy(mf_st, m_out.at[wid])
        pltpu.sync_copy(zf_st, z_out.at[wid])
        pltpu.sync_copy(if_st, i_out.at[wid])

    return sc_kern


def _uniform_at(flat_idx):
    x1 = flat_idx.astype(jnp.uint32) + _KS1
    u32 = lambda c: jnp.uint32(c)
    x0 = x1
    x1 = ((x1 << u32(_ROT_A[0])) | (x1 >> u32(32 - _ROT_A[0]))) ^ x0

    def rnds(x0, x1, rots, skip_first=False):
        for r in rots:
            x0 = x0 + x1
            x1 = ((x1 << u32(r)) | (x1 >> u32(32 - r))) ^ x0
        return x0, x1

    x0, x1 = rnds(x0, x1, _ROT_A[1:])
    x0, x1 = x0 + _KS1, x1 + u32(_KS2 + np.uint32(1))
    x0, x1 = rnds(x0, x1, _ROT_B)
    x0, x1 = x0 + _KS2, x1 + u32(_KS0 + np.uint32(2))
    x0, x1 = rnds(x0, x1, _ROT_A)
    x0, x1 = x0 + _KS0, x1 + u32(_KS1 + np.uint32(3))
    x0, x1 = rnds(x0, x1, _ROT_B)
    x0, x1 = x0 + _KS1, x1 + u32(_KS2 + np.uint32(4))
    x0, x1 = rnds(x0, x1, _ROT_A)
    x0, x1 = x0 + _KS2, x1 + u32(_KS0 + np.uint32(5))
    bits = x0 ^ x1
    fb = (bits >> u32(9)) | u32(0x3F800000)
    f = jax.lax.bitcast_convert_type(fb, jnp.float32)
    return jnp.maximum(_TINY, (f - np.float32(1.0)) + _TINY)


def _finish(logits, m_l, z_l, i_l):
    rows, vocab = logits.shape
    m = jnp.max(m_l, axis=1)
    z = jnp.sum(z_l * jnp.exp(m_l - m[:, None]), axis=1)
    cx = jnp.take_along_axis(logits, i_l, axis=1)
    flat = jnp.arange(rows, dtype=jnp.int32)[:, None] * vocab + i_l
    cu = _uniform_at(flat)
    g = -jnp.log(-jnp.log(cu))
    zscore = g + jnp.log(jnp.exp(cx - m[:, None]) / z[:, None])
    zbest = jnp.max(zscore, axis=1, keepdims=True)
    best = jnp.min(jnp.where(zscore == zbest, i_l, jnp.int32(2**30)), axis=1)
    return best.astype(jnp.int32)


_CSC = 9 * _CH


def kernel(logits):
    rows, vocab = logits.shape
    csc = _CSC if (vocab > 2 * _CSC and rows == 32) else 0
    if csc:
        m_sc, z_sc, i_sc = _make_sc_shard(rows, vocab, csc)(logits)
        m_tc, z_tc, i_tc = _run_tc(logits, width=8192, col0=csc)
        m_l = jnp.concatenate([m_tc, m_sc], axis=1)
        z_l = jnp.concatenate([z_tc, z_sc], axis=1)
        i_l = jnp.concatenate([i_tc, i_sc], axis=1)
    else:
        m_l, z_l, i_l = _run_tc(logits, width=8192)
    return _finish(logits, m_l, z_l, i_l)


# speedup vs baseline: 1.1248x; 1.1248x over previous
import functools

import jax
import jax.numpy as jnp
import numpy as np
from jax import lax
from jax.experimental import pallas as pl
from jax.experimental.pallas import tpu as pltpu
from jax.experimental.pallas import tpu_sc as plsc

_TINY = np.float32(np.finfo(np.float32).tiny)
_KS0 = np.uint32(0)
_KS1 = np.uint32(42)
_KS2 = np.uint32(_KS0 ^ _KS1 ^ np.uint32(0x1BD11BDA))
_ROT_A = (13, 15, 26, 6)
_ROT_B = (17, 29, 16, 24)
_LANES = 128


def _rotl(x, r):
    return jax.lax.shift_left(x, np.uint32(r)) | jax.lax.shift_right_logical(
        x, np.uint32(32 - r))


def _threefry_bits(x1):
    x0 = x1
    x1 = _rotl(x1, _ROT_A[0]) ^ x0
    for r in _ROT_A[1:]:
        x0 = x0 + x1
        x1 = _rotl(x1, r) ^ x0
    x0 = x0 + _KS1
    x1 = x1 + np.uint32(_KS2 + np.uint32(1))
    for r in _ROT_B:
        x0 = x0 + x1
        x1 = _rotl(x1, r) ^ x0
    x0 = x0 + _KS2
    x1 = x1 + np.uint32(_KS0 + np.uint32(2))
    for r in _ROT_A:
        x0 = x0 + x1
        x1 = _rotl(x1, r) ^ x0
    x0 = x0 + _KS0
    x1 = x1 + np.uint32(_KS1 + np.uint32(3))
    for r in _ROT_B:
        x0 = x0 + x1
        x1 = _rotl(x1, r) ^ x0
    x0 = x0 + _KS1
    x1 = x1 + np.uint32(_KS2 + np.uint32(4))
    for r in _ROT_A:
        x0 = x0 + x1
        x1 = _rotl(x1, r) ^ x0
    x0 = x0 + _KS2
    x1 = x1 + np.uint32(_KS0 + np.uint32(5))
    return x0 ^ x1


def _uniform_from_bits(bits):
    fb = jax.lax.shift_right_logical(bits, np.uint32(9)) | np.uint32(0x3F800000)
    f = jax.lax.bitcast_convert_type(fb, jnp.float32)
    return jnp.maximum(_TINY, (f - np.float32(1.0)) + _TINY)


_SW = 512


def _tc_kernel(logits_ref, m_out, z_out, i_out, m_ref, z_ref, tc_ref, wc_ref,
               i_ref, *, width, vocab, rows, nblocks, col0, colend, masked):
    j = pl.program_id(0)

    @pl.when(j == 0)
    def _init():
        m_ref[...] = jnp.full((rows, _SW), -np.inf, dtype=jnp.float32)
        z_ref[...] = jnp.zeros((rows, _SW), dtype=jnp.float32)
        tc_ref[...] = jnp.zeros((rows, _SW), dtype=jnp.float32)
        wc_ref[...] = jnp.full((rows, _SW), -1.0, dtype=jnp.float32)
        i_ref[...] = jnp.zeros((rows, _SW), dtype=jnp.int32)

    m_acc = m_ref[...]
    z_acc = z_ref[...]
    tc_acc = tc_ref[...]
    wc_acc = wc_ref[...]
    i_acc = i_ref[...]
    for k in range(width // _SW):
        xk_raw = logits_ref[:, k * _SW:(k + 1) * _SW]
        shape = xk_raw.shape
        col = (col0 + j * width + k * _SW
               + jax.lax.broadcasted_iota(jnp.int32, shape, 1))
        xk = jnp.where(col < colend, xk_raw, -jnp.inf) if masked else xk_raw
        row = jax.lax.broadcasted_iota(jnp.int32, shape, 0)
        ctr = (row * vocab + col + jnp.int32(_KS1)).astype(jnp.uint32)
        u = _uniform_from_bits(_threefry_bits(ctr))
        w = jnp.log(u)

        m_new = jnp.maximum(m_acc, xk)
        resc = jnp.exp(m_acc - m_new)
        t = jnp.exp(xk - m_new)
        z_acc = z_acc * resc + t
        tc_resc = tc_acc * resc
        better = (t * wc_acc) < (tc_resc * w)
        tc_acc = jnp.where(better, t, tc_resc)
        wc_acc = jnp.where(better, w, wc_acc)
        i_acc = jnp.where(better, col, i_acc)
        m_acc = m_new
    m_ref[...] = m_acc
    z_ref[...] = z_acc
    tc_ref[...] = tc_acc
    wc_ref[...] = wc_acc
    i_ref[...] = i_acc

    @pl.when(j == nblocks - 1)
    def _fin():
        m_out[...] = m_acc
        z_out[...] = z_acc
        i_out[...] = i_acc


def _run_tc(logits, width, col0=0, colend=None):
    rows, vocab = logits.shape
    if colend is None:
        colend = vocab
    nblocks = (colend - col0 + width - 1) // width
    boff = col0 // width
    masked = (colend - col0) % width != 0
    kern = functools.partial(_tc_kernel, width=width, vocab=vocab, rows=rows,
                             nblocks=nblocks, col0=col0, colend=colend,
                             masked=masked)
    acc = pl.pallas_call(
        kern,
        grid=(nblocks,),
        in_specs=[pl.BlockSpec((rows, width), lambda j: (0, j + boff))],
        out_specs=[pl.BlockSpec((rows, _SW), lambda j: (0, 0))] * 3,
        out_shape=[jax.ShapeDtypeStruct((rows, _SW), jnp.float32),
                   jax.ShapeDtypeStruct((rows, _SW), jnp.float32),
                   jax.ShapeDtypeStruct((rows, _SW), jnp.int32)],
        scratch_shapes=[pltpu.VMEM((rows, _SW), jnp.float32),
                        pltpu.VMEM((rows, _SW), jnp.float32),
                        pltpu.VMEM((rows, _SW), jnp.float32),
                        pltpu.VMEM((rows, _SW), jnp.float32),
                        pltpu.VMEM((rows, _SW), jnp.int32)],
        compiler_params=pltpu.CompilerParams(
            dimension_semantics=("arbitrary",)),
    )(logits)
    return acc


_CH = 28672
_LN2 = np.float32(0.6931471805599453)
_LOG1P_C = [np.float32((1.0 if k % 2 else -1.0) / k) for k in range(11, 0, -1)]


def _softlog(u):
    i = jax.lax.bitcast_convert_type(u, jnp.int32)
    e = jax.lax.shift_right_arithmetic(i, jnp.int32(23)) - jnp.int32(127)
    mb = (i & jnp.int32(0x007FFFFF)) | jnp.int32(0x3F800000)
    m = jax.lax.bitcast_convert_type(mb, jnp.float32)
    big = mb > jnp.int32(0x3FB504F3)
    m2 = jnp.where(big, m * np.float32(0.5), m)
    e2 = jnp.where(big, e + jnp.int32(1), e)
    y = m2 - np.float32(1.0)
    p = jnp.full_like(y, _LOG1P_C[0])
    for cns in _LOG1P_C[1:]:
        p = p * y + cns
    return e2.astype(jnp.float32) * _LN2 + y * p


def _make_sc_shard(rows, vocab, csc, tail0, tailn):
    nchunks = csc // _CH
    mesh = plsc.VectorSubcoreMesh(core_axis_name="c", subcore_axis_name="s")
    scratch = [pltpu.VMEM((2, _CH), jnp.float32),
               pltpu.VMEM((32,), jnp.float32),
               pltpu.VMEM((32,), jnp.float32),
               pltpu.VMEM((32,), jnp.int32),
               pltpu.SemaphoreType.DMA,
               pltpu.SemaphoreType.DMA]
    if tailn:
        scratch += [pltpu.VMEM((tailn,), jnp.float32),
                    pltpu.SemaphoreType.DMA]

    @functools.partial(
        pl.kernel, mesh=mesh,
        out_type=[jax.ShapeDtypeStruct((rows, 32), jnp.float32),
                  jax.ShapeDtypeStruct((rows, 32), jnp.float32),
                  jax.ShapeDtypeStruct((rows, 32), jnp.int32)],
        scratch_types=scratch,
    )
    def sc_kern(logits_hbm, m_out, z_out, i_out, buf, mf_st, zf_st, if_st,
                sem0, sem1, *tail_scr):
        wid = lax.axis_index("s") * 2 + lax.axis_index("c")
        rowbase = wid * jnp.int32(vocab) + jnp.int32(int(_KS1))
        sems = (sem0, sem1)
        copies = [None, None]
        copies[0] = pltpu.async_copy(
            logits_hbm.at[wid, pl.ds(0, _CH)], buf.at[0], sems[0])
        tail_cp = None
        if tailn:
            tbuf, tsem = tail_scr
            tail_cp = pltpu.async_copy(
                logits_hbm.at[wid, pl.ds(tail0, tailn)], tbuf, tsem)
        lane = lax.iota(jnp.int32, 16)

        def make_step(bufref, b, colbase):
            def step(i, st):
                st0, st1 = st
                new = []
                for half, sth in ((0, st0), (1, st1)):
                    m_a, z_a, tc_a, wc_a, ic_a = sth
                    if b is None:
                        xk = bufref[pl.ds(i * 32 + half * 16, 16)]
                    else:
                        xk = bufref[b, pl.ds(i * 32 + half * 16, 16)]
                    col = (jnp.int32(colbase) + i * 32 + half * 16) + lane
                    ctr = (rowbase + col).astype(jnp.uint32)
                    u = _uniform_from_bits(_threefry_bits(ctr))
                    w = _softlog(u)
                    m_new = jnp.maximum(m_a, xk)
                    resc = jnp.exp(m_a - m_new)
                    t = jnp.exp(xk - m_new)
                    z_a = z_a * resc + t
                    tc_r = tc_a * resc
                    better = (t * wc_a) < (tc_r * w)
                    tc_a = jnp.where(better, t, tc_r)
                    wc_a = jnp.where(better, w, wc_a)
                    ic_a = jnp.where(better, col, ic_a)
                    new.append((m_new, z_a, tc_a, wc_a, ic_a))
                return (new[0], new[1])

            return step

        half_init = (jnp.full((16,), np.float32(-3.0e38), jnp.float32),
                     jnp.zeros((16,), jnp.float32),
                     jnp.zeros((16,), jnp.float32),
                     jnp.full((16,), np.float32(-1.0), jnp.float32),
                     jnp.zeros((16,), jnp.int32))
        carry = (half_init, half_init)
        for c in range(nchunks):
            b = c % 2
            if c + 1 < nchunks:
                copies[1 - b] = pltpu.async_copy(
                    logits_hbm.at[wid, pl.ds((c + 1) * _CH, _CH)],
                    buf.at[1 - b], sems[1 - b])
            copies[b].wait()
            carry = lax.fori_loop(0, _CH // 32, make_step(buf, b, c * _CH),
                                  carry)
        if tailn:
            tail_cp.wait()
            carry = lax.fori_loop(0, tailn // 32,
                                  make_step(tbuf, None, tail0), carry)
        for half in (0, 1):
            m_a, z_a, tc_a, wc_a, ic_a = carry[half]
            sl = pl.ds(half * 16, 16)
            mf_st[sl] = m_a
            zf_st[sl] = z_a
            if_st[sl] = ic_a
        pltpu.sync_copy(mf_st, m_out.at[wid])
        pltpu.sync_copy(zf_st, z_out.at[wid])
        pltpu.sync_copy(if_st, i_out.at[wid])

    return sc_kern


def _uniform_at(flat_idx):
    x1 = flat_idx.astype(jnp.uint32) + _KS1
    u32 = lambda c: jnp.uint32(c)
    x0 = x1
    x1 = ((x1 << u32(_ROT_A[0])) | (x1 >> u32(32 - _ROT_A[0]))) ^ x0

    def rnds(x0, x1, rots, skip_first=False):
        for r in rots:
            x0 = x0 + x1
            x1 = ((x1 << u32(r)) | (x1 >> u32(32 - r))) ^ x0
        return x0, x1

    x0, x1 = rnds(x0, x1, _ROT_A[1:])
    x0, x1 = x0 + _KS1, x1 + u32(_KS2 + np.uint32(1))
    x0, x1 = rnds(x0, x1, _ROT_B)
    x0, x1 = x0 + _KS2, x1 + u32(_KS0 + np.uint32(2))
    x0, x1 = rnds(x0, x1, _ROT_A)
    x0, x1 = x0 + _KS0, x1 + u32(_KS1 + np.uint32(3))
    x0, x1 = rnds(x0, x1, _ROT_B)
    x0, x1 = x0 + _KS1, x1 + u32(_KS2 + np.uint32(4))
    x0, x1 = rnds(x0, x1, _ROT_A)
    x0, x1 = x0 + _KS2, x1 + u32(_KS0 + np.uint32(5))
    bits = x0 ^ x1
    fb = (bits >> u32(9)) | u32(0x3F800000)
    f = jax.lax.bitcast_convert_type(fb, jnp.float32)
    return jnp.maximum(_TINY, (f - np.float32(1.0)) + _TINY)


def _finish(logits, m_l, z_l, i_l):
    rows, vocab = logits.shape
    m = jnp.max(m_l, axis=1)
    z = jnp.sum(z_l * jnp.exp(m_l - m[:, None]), axis=1)
    cx = jnp.take_along_axis(logits, i_l, axis=1)
    flat = jnp.arange(rows, dtype=jnp.int32)[:, None] * vocab + i_l
    cu = _uniform_at(flat)
    g = -jnp.log(-jnp.log(cu))
    zscore = g + jnp.log(jnp.exp(cx - m[:, None]) / z[:, None])
    zbest = jnp.max(zscore, axis=1, keepdims=True)
    best = jnp.min(jnp.where(zscore == zbest, i_l, jnp.int32(2**30)), axis=1)
    return best.astype(jnp.int32)


_CSC = 8 * _CH


def kernel(logits):
    rows, vocab = logits.shape
    csc = _CSC if (vocab > 2 * _CSC and rows == 32) else 0
    if csc:
        tailn = (vocab - csc) % 8192
        if tailn % 32 != 0 or (vocab - tailn) % 8 != 0:
            tailn = 0
        tail0 = vocab - tailn
        m_sc, z_sc, i_sc = _make_sc_shard(rows, vocab, csc, tail0,
                                          tailn)(logits)
        m_tc, z_tc, i_tc = _run_tc(logits, width=8192, col0=csc,
                                   colend=tail0)
        m_l = jnp.concatenate([m_tc, m_sc], axis=1)
        z_l = jnp.concatenate([z_tc, z_sc], axis=1)
        i_l = jnp.concatenate([i_tc, i_sc], axis=1)
    else:
        m_l, z_l, i_l = _run_tc(logits, width=8192)
    return _finish(logits, m_l, z_l, i_l)
